# baseline (speedup 1.0000x reference)
import functools
import jax
import jax.numpy as jnp
from jax import lax
from jax.experimental import pallas as pl
from jax.experimental.pallas import tpu as pltpu
from jax.experimental.pallas import tpu_sc as plsc

BATCH = 16384
EMB = 64
NC, NS = 2, 16
NW = NC * NS
B_PER_W = BATCH // NW
CH = 128
NCH = B_PER_W // CH
IDROWS = BATCH // CH

_sc_mesh = plsc.VectorSubcoreMesh(core_axis_name="c", subcore_axis_name="s")


@functools.partial(
    pl.kernel,
    out_type=[jax.ShapeDtypeStruct((BATCH, EMB), jnp.float32)] * 4,
    mesh=_sc_mesh,
    compiler_params=pltpu.CompilerParams(use_tc_tiling_on_sc=False),
    scratch_types=[
        pltpu.VMEM((NCH, CH), jnp.int32),
        pltpu.VMEM((NCH, CH), jnp.int32),
        pltpu.VMEM((B_PER_W, EMB), jnp.float32),
        pltpu.VMEM((B_PER_W, EMB), jnp.float32),
        pltpu.SemaphoreType.DMA,
        pltpu.SemaphoreType.DMA,
    ],
)
def _sc_gather(uid_hbm, iid_hbm, gu_hbm, gi_hbm, mu_hbm, mi_hbm,
               out_gu, out_gi, out_mu, out_mi,
               uidx, iidx, bufa, bufb, sema, semb):
    wid = lax.axis_index("s") * NC + lax.axis_index("c")
    base = wid * B_PER_W
    pltpu.sync_copy(uid_hbm.at[pl.ds(wid * NCH, NCH)], uidx)
    pltpu.sync_copy(iid_hbm.at[pl.ds(wid * NCH, NCH)], iidx)

    def gather_pair(idx, tab0, tab1, out0, out1):
        cps = []
        for j in range(NCH):
            cps.append(pltpu.async_copy(
                tab0.at[idx.at[j]], bufa.at[pl.ds(j * CH, CH)], sema))
        for j in range(NCH):
            cps.append(pltpu.async_copy(
                tab1.at[idx.at[j]], bufb.at[pl.ds(j * CH, CH)], semb))
        for cp in cps[:NCH]:
            cp.wait()
        pltpu.sync_copy(bufa, out0.at[pl.ds(base, B_PER_W)])
        for cp in cps[NCH:]:
            cp.wait()
        pltpu.sync_copy(bufb, out1.at[pl.ds(base, B_PER_W)])

    gather_pair(uidx, gu_hbm, mu_hbm, out_gu, out_mu)
    gather_pair(iidx, gi_hbm, mi_hbm, out_gi, out_mi)


BB = 4096


def _tc_mlp_body(gu, gi, mu, mi, w1a, w1b, b1, w2, b2, w3, b3, wog, woh, bo,
                 out):
    f32 = jnp.float32
    g = gu[:] * gi[:]
    acc = jnp.dot(g, wog[:], preferred_element_type=f32)
    h = jnp.dot(mu[:], w1a[:], preferred_element_type=f32)
    h = h + jnp.dot(mi[:], w1b[:], preferred_element_type=f32)
    h = jnp.maximum(h + b1[:], 0.0)
    h = jnp.maximum(jnp.dot(h, w2[:], preferred_element_type=f32) + b2[:], 0.0)
    h = jnp.maximum(jnp.dot(h, w3[:], preferred_element_type=f32) + b3[:], 0.0)
    out[:] = acc + jnp.dot(h, woh[:], preferred_element_type=f32) + bo[0, 0]


def _row_spec():
    return pl.BlockSpec((BB, EMB), lambda i: (i, 0))


def _full_spec(shape):
    return pl.BlockSpec(shape, lambda i: tuple(0 for _ in shape))


_tc_mlp = pl.pallas_call(
    _tc_mlp_body,
    grid=(BATCH // BB,),
    in_specs=[
        _row_spec(), _row_spec(), _row_spec(), _row_spec(),
        _full_spec((EMB, 128)), _full_spec((EMB, 128)), _full_spec((1, 128)),
        _full_spec((128, 64)), _full_spec((1, 64)),
        _full_spec((64, 32)), _full_spec((1, 32)),
        _full_spec((EMB, 1)), _full_spec((32, 1)), _full_spec((1, 1)),
    ],
    out_specs=pl.BlockSpec((BB, 1), lambda i: (i, 0)),
    out_shape=jax.ShapeDtypeStruct((BATCH, 1), jnp.float32),
)


@jax.jit
def kernel(user_ids, item_ids, gmf_user_emb, gmf_item_emb, mlp_user_emb,
           mlp_item_emb, W1, b1, W2, b2, W3, b3, Wo, bo):
    uid2d = user_ids.astype(jnp.int32).reshape(IDROWS, CH)
    iid2d = item_ids.astype(jnp.int32).reshape(IDROWS, CH)
    gu, gi, mu, mi = _sc_gather(uid2d, iid2d, gmf_user_emb, gmf_item_emb,
                                mlp_user_emb, mlp_item_emb)
    pred = _tc_mlp(gu, gi, mu, mi,
                   W1[:EMB], W1[EMB:], b1.reshape(1, -1),
                   W2, b2.reshape(1, -1), W3, b3.reshape(1, -1),
                   Wo[:EMB], Wo[EMB:], bo.reshape(1, 1))
    return pred.reshape(BATCH)


# speedup vs baseline: 1.1117x; 1.1117x over previous
import functools
import jax
import jax.numpy as jnp
from jax import lax
from jax.experimental import pallas as pl
from jax.experimental.pallas import tpu as pltpu
from jax.experimental.pallas import tpu_sc as plsc

BATCH = 16384
EMB = 64
NC, NS = 2, 16
NW = NC * NS
B_PER_W = BATCH // NW
CH = 128
NCH = B_PER_W // CH
ROWS_C = 256
IDROWS = BATCH // CH

_sc_mesh = plsc.VectorSubcoreMesh(core_axis_name="c", subcore_axis_name="s")


@functools.partial(
    pl.kernel,
    out_type=[jax.ShapeDtypeStruct((BATCH, 2 * EMB), jnp.float32)] * 2,
    mesh=_sc_mesh,
    compiler_params=pltpu.CompilerParams(use_tc_tiling_on_sc=False),
    scratch_types=[
        pltpu.VMEM((NCH, CH), jnp.int32),
        pltpu.VMEM((NCH, CH), jnp.int32),
        pltpu.VMEM((ROWS_C, EMB), jnp.float32),
        pltpu.VMEM((ROWS_C, EMB), jnp.float32),
        pltpu.VMEM((ROWS_C, EMB), jnp.float32),
        pltpu.VMEM((ROWS_C, EMB), jnp.float32),
        pltpu.SemaphoreType.DMA,
        pltpu.SemaphoreType.DMA,
    ],
)
def _sc_gather(uid_hbm, iid_hbm, gu_hbm, gi_hbm, mu_hbm, mi_hbm,
               out_u, out_i, uidx, iidx, bgu, bmu, bgi, bmi, sema, semb):
    wid = lax.axis_index("s") * NC + lax.axis_index("c")
    base = wid * B_PER_W
    pltpu.sync_copy(uid_hbm.at[pl.ds(wid * NCH, NCH)], uidx)
    pltpu.sync_copy(iid_hbm.at[pl.ds(wid * NCH, NCH)], iidx)

    for c in range(B_PER_W // ROWS_C):
        cps = []
        for j in range(ROWS_C // CH):
            k = c * (ROWS_C // CH) + j
            dst_rows = pl.ds(j * CH, CH)
            cps.append(pltpu.async_copy(
                gu_hbm.at[uidx.at[k]], bgu.at[dst_rows], sema))
            cps.append(pltpu.async_copy(
                mu_hbm.at[uidx.at[k]], bmu.at[dst_rows], sema))
            cps.append(pltpu.async_copy(
                gi_hbm.at[iidx.at[k]], bgi.at[dst_rows], semb))
            cps.append(pltpu.async_copy(
                mi_hbm.at[iidx.at[k]], bmi.at[dst_rows], semb))
        for cp in cps:
            cp.wait()
        orows = pl.ds(base + c * ROWS_C, ROWS_C)
        pltpu.sync_copy(bgu, out_u.at[orows, pl.ds(0, EMB)])
        pltpu.sync_copy(bmu, out_u.at[orows, pl.ds(EMB, EMB)])
        pltpu.sync_copy(bgi, out_i.at[orows, pl.ds(0, EMB)])
        pltpu.sync_copy(bmi, out_i.at[orows, pl.ds(EMB, EMB)])


BB = 4096


def _tc_mlp_body(u, it, w1a, w1b, b1, w2, b2, w3, b3, wog, woh, bo, out):
    f32 = jnp.float32
    uu = u[:]
    ii = it[:]
    g = uu[:, :EMB] * ii[:, :EMB]
    acc = jnp.dot(g, wog[:], preferred_element_type=f32)
    h = jnp.dot(uu[:, EMB:], w1a[:], preferred_element_type=f32)
    h = h + jnp.dot(ii[:, EMB:], w1b[:], preferred_element_type=f32)
    h = jnp.maximum(h + b1[:], 0.0)
    h = jnp.maximum(jnp.dot(h, w2[:], preferred_element_type=f32) + b2[:], 0.0)
    h = jnp.maximum(jnp.dot(h, w3[:], preferred_element_type=f32) + b3[:], 0.0)
    out[:] = acc + jnp.dot(h, woh[:], preferred_element_type=f32) + bo[0, 0]


def _row_spec():
    return pl.BlockSpec((BB, 2 * EMB), lambda i: (i, 0))


def _full_spec(shape):
    return pl.BlockSpec(shape, lambda i: tuple(0 for _ in shape))


_tc_mlp = pl.pallas_call(
    _tc_mlp_body,
    grid=(BATCH // BB,),
    in_specs=[
        _row_spec(), _row_spec(),
        _full_spec((EMB, 128)), _full_spec((EMB, 128)), _full_spec((1, 128)),
        _full_spec((128, 64)), _full_spec((1, 64)),
        _full_spec((64, 32)), _full_spec((1, 32)),
        _full_spec((EMB, 1)), _full_spec((32, 1)), _full_spec((1, 1)),
    ],
    out_specs=pl.BlockSpec((BB, 1), lambda i: (i, 0)),
    out_shape=jax.ShapeDtypeStruct((BATCH, 1), jnp.float32),
)


@jax.jit
def kernel(user_ids, item_ids, gmf_user_emb, gmf_item_emb, mlp_user_emb,
           mlp_item_emb, W1, b1, W2, b2, W3, b3, Wo, bo):
    uid2d = user_ids.astype(jnp.int32).reshape(IDROWS, CH)
    iid2d = item_ids.astype(jnp.int32).reshape(IDROWS, CH)
    rows_u, rows_i = _sc_gather(uid2d, iid2d, gmf_user_emb, gmf_item_emb,
                                mlp_user_emb, mlp_item_emb)
    pred = _tc_mlp(rows_u, rows_i,
                   W1[:EMB], W1[EMB:], b1.reshape(1, -1),
                   W2, b2.reshape(1, -1), W3, b3.reshape(1, -1),
                   Wo[:EMB], Wo[EMB:], bo.reshape(1, 1))
    return pred.reshape(BATCH)


# speedup vs baseline: 1.7082x; 1.5366x over previous
import functools
import jax
import jax.numpy as jnp
from jax import lax
from jax.experimental import pallas as pl
from jax.experimental.pallas import tpu as pltpu
from jax.experimental.pallas import tpu_sc as plsc

BATCH = 16384
EMB = 64
NC, NS = 2, 16
NW = NC * NS
B_PER_W = BATCH // NW
CH = 128
NCH = B_PER_W // CH
IDROWS = BATCH // CH

_sc_mesh = plsc.VectorSubcoreMesh(core_axis_name="c", subcore_axis_name="s")


@functools.partial(
    pl.kernel,
    out_type=[jax.ShapeDtypeStruct((BATCH, 2 * EMB), jnp.float32)] * 2,
    mesh=_sc_mesh,
    compiler_params=pltpu.CompilerParams(use_tc_tiling_on_sc=True),
    scratch_types=[
        pltpu.VMEM((NCH, CH), jnp.int32),
        pltpu.VMEM((NCH, CH), jnp.int32),
        pltpu.VMEM((CH, 2 * EMB), jnp.float32),
        pltpu.VMEM((CH, 2 * EMB), jnp.float32),
        pltpu.VMEM((CH, 2 * EMB), jnp.float32),
        pltpu.VMEM((CH, 2 * EMB), jnp.float32),
        pltpu.SemaphoreType.DMA,
        pltpu.SemaphoreType.DMA,
    ],
)
def _sc_gather(uid_hbm, iid_hbm, utab_hbm, itab_hbm,
               out_u, out_i, uidx, iidx, bu0, bu1, bi0, bi1, semg, semw):
    wid = lax.axis_index("s") * NC + lax.axis_index("c")
    base = wid * B_PER_W
    pltpu.sync_copy(uid_hbm.at[pl.ds(wid * NCH, NCH)], uidx)
    pltpu.sync_copy(iid_hbm.at[pl.ds(wid * NCH, NCH)], iidx)

    ubufs, ibufs = (bu0, bu1), (bi0, bi1)
    gath = []
    for k in range(NCH):
        bu, bi = ubufs[k % 2], ibufs[k % 2]
        gath.append((
            pltpu.async_copy(utab_hbm.at[uidx.at[k]], bu, semg),
            pltpu.async_copy(itab_hbm.at[iidx.at[k]], bi, semg),
        ))
        if k >= 1:
            pbu, pbi = ubufs[(k - 1) % 2], ibufs[(k - 1) % 2]
            for cp in gath[k - 1]:
                cp.wait()
            orows = pl.ds(base + (k - 1) * CH, CH)
            pltpu.async_copy(pbu, out_u.at[orows], semw)
            pltpu.async_copy(pbi, out_i.at[orows], semw)
    for cp in gath[NCH - 1]:
        cp.wait()
    orows = pl.ds(base + (NCH - 1) * CH, CH)
    wu = pltpu.async_copy(ubufs[(NCH - 1) % 2], out_u.at[orows], semw)
    wi = pltpu.async_copy(ibufs[(NCH - 1) % 2], out_i.at[orows], semw)
    for k in range(NCH - 1):
        orows = pl.ds(base + k * CH, CH)
        pltpu.make_async_copy(ubufs[k % 2], out_u.at[orows], semw).wait()
        pltpu.make_async_copy(ibufs[k % 2], out_i.at[orows], semw).wait()
    wu.wait()
    wi.wait()


BB = 4096


def _tc_mlp_body(u, it, w1a, w1b, b1, w2, b2, w3, b3, wog, woh, bo, out):
    f32 = jnp.float32
    uu = u[:]
    ii = it[:]
    g = uu[:, :EMB] * ii[:, :EMB]
    acc = jnp.dot(g, wog[:], preferred_element_type=f32)
    h = jnp.dot(uu[:, EMB:], w1a[:], preferred_element_type=f32)
    h = h + jnp.dot(ii[:, EMB:], w1b[:], preferred_element_type=f32)
    h = jnp.maximum(h + b1[:], 0.0)
    h = jnp.maximum(jnp.dot(h, w2[:], preferred_element_type=f32) + b2[:], 0.0)
    h = jnp.maximum(jnp.dot(h, w3[:], preferred_element_type=f32) + b3[:], 0.0)
    out[:] = acc + jnp.dot(h, woh[:], preferred_element_type=f32) + bo[0, 0]


def _row_spec():
    return pl.BlockSpec((BB, 2 * EMB), lambda i: (i, 0))


def _full_spec(shape):
    return pl.BlockSpec(shape, lambda i: tuple(0 for _ in shape))


_tc_mlp = pl.pallas_call(
    _tc_mlp_body,
    grid=(BATCH // BB,),
    in_specs=[
        _row_spec(), _row_spec(),
        _full_spec((EMB, 128)), _full_spec((EMB, 128)), _full_spec((1, 128)),
        _full_spec((128, 64)), _full_spec((1, 64)),
        _full_spec((64, 32)), _full_spec((1, 32)),
        _full_spec((EMB, 1)), _full_spec((32, 1)), _full_spec((1, 1)),
    ],
    out_specs=pl.BlockSpec((BB, 1), lambda i: (i, 0)),
    out_shape=jax.ShapeDtypeStruct((BATCH, 1), jnp.float32),
)


@jax.jit
def kernel(user_ids, item_ids, gmf_user_emb, gmf_item_emb, mlp_user_emb,
           mlp_item_emb, W1, b1, W2, b2, W3, b3, Wo, bo):
    uid2d = user_ids.astype(jnp.int32).reshape(IDROWS, CH)
    iid2d = item_ids.astype(jnp.int32).reshape(IDROWS, CH)
    utab = jnp.concatenate([gmf_user_emb, mlp_user_emb], axis=1)
    itab = jnp.concatenate([gmf_item_emb, mlp_item_emb], axis=1)
    rows_u, rows_i = _sc_gather(uid2d, iid2d, utab, itab)
    pred = _tc_mlp(rows_u, rows_i,
                   W1[:EMB], W1[EMB:], b1.reshape(1, -1),
                   W2, b2.reshape(1, -1), W3, b3.reshape(1, -1),
                   Wo[:EMB], Wo[EMB:], bo.reshape(1, 1))
    return pred.reshape(BATCH)
